# Initial kernel scaffold; baseline (speedup 1.0000x reference)
import jax
import jax.numpy as jnp
from jax.experimental import pallas as pl


def kernel(coordinates, active, occupancies, lmax, radial_densities, grid_to_cartesian, out):
    raise NotImplementedError("write your pallas kernel here")


# speedup vs baseline: 40.9162x; 40.9162x over previous
import functools

import jax
import jax.numpy as jnp
from jax import lax
from jax.experimental import pallas as pl
from jax.experimental.pallas import tpu as pltpu
from jax.experimental.pallas import tpu_sc as plsc

_RMAX = 3.0
_RSTEP = 0.01
_LBOX = 13
_L3 = _LBOX * _LBOX * _LBOX
_NCHUNK = 18
_PTS = _NCHUNK * 128
_LANES = 16
_NC, _NS = 2, 16
_NPF = 10


def _mod(a, m):
    r = lax.rem(a, m)
    return jnp.where(r < 0, r + m, r)


def _bf16_round(v):
    u = lax.bitcast_convert_type(v, jnp.int32)
    lsb = lax.shift_right_logical(u, 16) & 1
    u = (u + 0x7FFF + lsb) & jnp.int32(-65536)
    return lax.bitcast_convert_type(u, jnp.float32)


def _sc_body(tabf_hbm, tabi_hbm, pf_hbm, m_hbm, rad_hbm, seed_hbm, out_hbm,
             tabf_v, tabi_v, pf_v, m_v, rad_v, val_v, idx_v, grid_sh, sem,
             *, apw, nflat_sc, gz, gy, gx, nrad_m1):
    cid = lax.axis_index("c")
    sid = lax.axis_index("s")
    wid = cid * _NS + sid
    share = nflat_sc // _NS

    pltpu.sync_copy(seed_hbm.at[cid, pl.ds(sid * share, share)],
                    grid_sh.at[pl.ds(sid * share, share)])

    pltpu.sync_copy(tabf_hbm, tabf_v)
    pltpu.sync_copy(tabi_hbm, tabi_v)
    pltpu.sync_copy(pf_hbm.at[wid], pf_v)
    pltpu.sync_copy(m_hbm, m_v)
    plsc.subcore_barrier()

    g00 = m_v[pl.ds(0, _LANES)]
    g01 = m_v[pl.ds(16, _LANES)]
    g02 = m_v[pl.ds(32, _LANES)]
    g10 = m_v[pl.ds(48, _LANES)]
    g11 = m_v[pl.ds(64, _LANES)]
    g12 = m_v[pl.ds(80, _LANES)]
    g20 = m_v[pl.ds(96, _LANES)]
    g21 = m_v[pl.ds(112, _LANES)]
    g22 = m_v[pl.ds(128, _LANES)]
    rmax2 = jnp.full((_LANES,), _RMAX * _RMAX, jnp.float32)
    rstep = jnp.full((_LANES,), _RSTEP, jnp.float32)
    half = jnp.full((_LANES,), 0.5, jnp.float32)
    onep5 = jnp.full((_LANES,), 1.5, jnp.float32)
    magic = jnp.full((_LANES,), 0x5F3759DF, jnp.int32)

    def atom_body(a, _):
        gid = wid * apw + a
        pltpu.sync_copy(rad_hbm.at[gid], rad_v)

        cz = pf_v[a, pl.ds(0, _LANES)]
        cy = pf_v[a, pl.ds(16, _LANES)]
        cx = pf_v[a, pl.ds(32, _LANES)]
        bzf = pf_v[a, pl.ds(48, _LANES)]
        byf = pf_v[a, pl.ds(64, _LANES)]
        bxf = pf_v[a, pl.ds(80, _LANES)]
        bz = bzf.astype(jnp.int32)
        by = byf.astype(jnp.int32)
        bx = bxf.astype(jnp.int32)
        ez = pf_v[a, pl.ds(96, _LANES)]
        ey = pf_v[a, pl.ds(112, _LANES)]
        ex = pf_v[a, pl.ds(128, _LANES)]
        w = pf_v[a, pl.ds(144, _LANES)]

        def row_body(j, _):
            for cc in range(128 // _LANES):
                base = j * 128 + cc * _LANES
                ozf = tabf_v[0, pl.ds(base, _LANES)]
                oyf = tabf_v[1, pl.ds(base, _LANES)]
                oxf = tabf_v[2, pl.ds(base, _LANES)]
                ozi = tabi_v[0, pl.ds(base, _LANES)]
                oyi = tabi_v[1, pl.ds(base, _LANES)]
                oxi = tabi_v[2, pl.ds(base, _LANES)]

                rz = _bf16_round((bzf + ozf) - cz)
                ry = _bf16_round((byf + oyf) - cy)
                rx = _bf16_round((bxf + oxf) - cx)
                c0 = (rz * g00 + ry * g01) + rx * g02
                c1 = (rz * g10 + ry * g11) + rx * g12
                c2 = (rz * g20 + ry * g21) + rx * g22
                d2 = (c0 * c0 + c1 * c1) + c2 * c2
                x = jnp.maximum(d2, jnp.float32(1e-12))
                i = lax.bitcast_convert_type(x, jnp.int32)
                y = lax.bitcast_convert_type(
                    magic - lax.shift_right_arithmetic(i, 1), jnp.float32)
                hx = half * x
                y = y * (onep5 - hx * y * y)
                y = y * (onep5 - hx * y * y)
                y = y * (onep5 - hx * y * y)
                d = x * y
                ridx = jnp.clip((d / rstep).astype(jnp.int32), 0, nrad_m1)
                dens = plsc.load_gather(rad_v, [ridx])
                mask = ((d2 <= rmax2) & (ozf <= ez)
                        & (oyf <= ey) & (oxf <= ex))
                val = jnp.where(mask, dens * w, jnp.float32(0.0))

                zi = _mod(bz + ozi, gz)
                yi = _mod(by + oyi, gy)
                xi = _mod(bx + oxi, gx)
                flat = (zi * gy + yi) * gx + xi

                val_v[pl.ds(base, _LANES)] = val
                idx_v[j, pl.ds(cc * _LANES, _LANES)] = flat
            return 0

        lax.fori_loop(0, _NCHUNK, row_body, 0)

        for j in range(_NCHUNK):
            pltpu.sync_copy(val_v.at[pl.ds(j * 128, 128)],
                            grid_sh.at[idx_v.at[j]], add=True)
        return 0

    lax.fori_loop(0, apw, atom_body, 0)

    plsc.subcore_barrier()
    pltpu.sync_copy(grid_sh.at[pl.ds(sid * share, share)],
                    out_hbm.at[cid, pl.ds(sid * share, share)])


def kernel(coordinates, active, occupancies, lmax, radial_densities,
           grid_to_cartesian, out):
    b, n = coordinates.shape[:2]
    bn = b * n
    gz, gy, gx = out.shape[1], out.shape[2], out.shape[3]
    nflat_sc = gz * gy * gx
    nrad = radial_densities.shape[-1]
    nrad_pad = ((nrad + 127) // 128) * 128
    apw = bn // (_NC * _NS)

    f32 = jnp.float32
    coords = coordinates.reshape(bn, 3)
    b_min = jnp.ceil(coords - lmax[None, :]).astype(jnp.int32)
    b_max = jnp.floor(coords + lmax[None, :]).astype(jnp.int32)
    ext = (b_max - b_min).astype(f32)
    w = occupancies.reshape(bn) * active.reshape(bn).astype(f32)

    params = jnp.stack([
        coords[:, 0], coords[:, 1], coords[:, 2],
        b_min[:, 0].astype(f32), b_min[:, 1].astype(f32),
        b_min[:, 2].astype(f32),
        ext[:, 0], ext[:, 1], ext[:, 2],
        w,
    ], axis=1)
    pf = jnp.broadcast_to(params[:, :, None], (bn, _NPF, _LANES))
    pf = jnp.pad(pf.reshape(bn, _NPF * _LANES),
                 ((0, 0), (0, 256 - _NPF * _LANES)))
    pf = pf.reshape(_NC * _NS, apw, 256)

    gb = grid_to_cartesian.astype(jnp.bfloat16).astype(f32)
    m_in = jnp.pad(
        jnp.broadcast_to(gb.reshape(9)[:, None], (9, _LANES)).reshape(144),
        (0, 112))

    t = jnp.arange(_PTS, dtype=jnp.int32)
    oz = jnp.where(t < _L3, t // (_LBOX * _LBOX), _LBOX)
    oy = jnp.where(t < _L3, (t // _LBOX) % _LBOX, _LBOX)
    ox = jnp.where(t < _L3, t % _LBOX, _LBOX)
    tabi = jnp.stack([oz, oy, ox])
    tabf = tabi.astype(f32)

    rad = radial_densities.reshape(bn, nrad)
    rad = jnp.pad(rad, ((0, 0), (0, nrad_pad - nrad)))

    seed = out.reshape(b, nflat_sc)

    sc = functools.partial(
        pl.kernel,
        out_type=jax.ShapeDtypeStruct((b, nflat_sc), f32),
        mesh=plsc.VectorSubcoreMesh(core_axis_name="c", subcore_axis_name="s",
                                    num_cores=_NC, num_subcores=_NS),
        compiler_params=pltpu.CompilerParams(needs_layout_passes=False),
        scratch_types=[
            pltpu.VMEM((3, _PTS), f32),
            pltpu.VMEM((3, _PTS), jnp.int32),
            pltpu.VMEM((apw, 256), f32),
            pltpu.VMEM((256,), f32),
            pltpu.VMEM((nrad_pad,), f32),
            pltpu.VMEM((_PTS,), f32),
            pltpu.VMEM((_NCHUNK, 128), jnp.int32),
            pltpu.VMEM_SHARED((nflat_sc,), f32),
            pltpu.SemaphoreType.DMA,
        ],
    )(functools.partial(_sc_body, apw=apw, nflat_sc=nflat_sc,
                        gz=gz, gy=gy, gx=gx, nrad_m1=nrad - 1))

    res = sc(tabf, tabi, pf, m_in, rad, seed)
    return res.reshape(out.shape)


# speedup vs baseline: 42.4450x; 1.0374x over previous
import functools

import jax
import jax.numpy as jnp
from jax import lax
from jax.experimental import pallas as pl
from jax.experimental.pallas import tpu as pltpu
from jax.experimental.pallas import tpu_sc as plsc

_RMAX = 3.0
_RSTEP = 0.01
_LBOX = 13
_L3 = _LBOX * _LBOX * _LBOX
_NCHUNK = 18
_PTS = _NCHUNK * 128
_LANES = 16
_NC, _NS = 2, 16
_NPF = 10


def _mod(a, m):
    r = lax.rem(a, m)
    return jnp.where(r < 0, r + m, r)


def _bf16_round(v):
    u = lax.bitcast_convert_type(v, jnp.int32)
    lsb = lax.shift_right_logical(u, 16) & 1
    u = (u + 0x7FFF + lsb) & jnp.int32(-65536)
    return lax.bitcast_convert_type(u, jnp.float32)


def _sc_body(tabf_hbm, tabi_hbm, pf_hbm, m_hbm, rad_hbm, seed_hbm, out_hbm,
             tabf_v, tabi_v, pf_v, m_v, rad_v, val_v, idx_v, grid_sh, sem,
             *, apw, nflat_sc, gz, gy, gx, nrad_m1):
    cid = lax.axis_index("c")
    sid = lax.axis_index("s")
    wid = cid * _NS + sid
    share = nflat_sc // _NS

    pltpu.sync_copy(seed_hbm.at[cid, pl.ds(sid * share, share)],
                    grid_sh.at[pl.ds(sid * share, share)])

    pltpu.sync_copy(tabf_hbm, tabf_v)
    pltpu.sync_copy(tabi_hbm, tabi_v)
    pltpu.sync_copy(pf_hbm.at[wid], pf_v)
    pltpu.sync_copy(m_hbm, m_v)
    plsc.subcore_barrier()

    g00 = m_v[pl.ds(0, _LANES)]
    g01 = m_v[pl.ds(16, _LANES)]
    g02 = m_v[pl.ds(32, _LANES)]
    g10 = m_v[pl.ds(48, _LANES)]
    g11 = m_v[pl.ds(64, _LANES)]
    g12 = m_v[pl.ds(80, _LANES)]
    g20 = m_v[pl.ds(96, _LANES)]
    g21 = m_v[pl.ds(112, _LANES)]
    g22 = m_v[pl.ds(128, _LANES)]
    rmax2 = jnp.full((_LANES,), _RMAX * _RMAX, jnp.float32)
    rstep_inv = jnp.full((_LANES,), jnp.float32(1.0) / jnp.float32(_RSTEP),
                         jnp.float32)
    half = jnp.full((_LANES,), 0.5, jnp.float32)
    onep5 = jnp.full((_LANES,), 1.5, jnp.float32)
    magic = jnp.full((_LANES,), 0x5F3759DF, jnp.int32)

    def atom_body(a, _):
        gid = wid * apw + a
        pltpu.sync_copy(rad_hbm.at[gid], rad_v)

        cz = pf_v[a, pl.ds(0, _LANES)]
        cy = pf_v[a, pl.ds(16, _LANES)]
        cx = pf_v[a, pl.ds(32, _LANES)]
        bzf = pf_v[a, pl.ds(48, _LANES)]
        byf = pf_v[a, pl.ds(64, _LANES)]
        bxf = pf_v[a, pl.ds(80, _LANES)]
        bz = bzf.astype(jnp.int32)
        by = byf.astype(jnp.int32)
        bx = bxf.astype(jnp.int32)
        ez = pf_v[a, pl.ds(96, _LANES)]
        ey = pf_v[a, pl.ds(112, _LANES)]
        ex = pf_v[a, pl.ds(128, _LANES)]
        w = pf_v[a, pl.ds(144, _LANES)]

        def row_body(j, _):
            for cc in range(128 // _LANES):
                base = j * 128 + cc * _LANES
                ozf = tabf_v[0, pl.ds(base, _LANES)]
                oyf = tabf_v[1, pl.ds(base, _LANES)]
                oxf = tabf_v[2, pl.ds(base, _LANES)]
                ozi = tabi_v[0, pl.ds(base, _LANES)]
                oyi = tabi_v[1, pl.ds(base, _LANES)]
                oxi = tabi_v[2, pl.ds(base, _LANES)]

                rz = _bf16_round((bzf + ozf) - cz)
                ry = _bf16_round((byf + oyf) - cy)
                rx = _bf16_round((bxf + oxf) - cx)
                c0 = (rz * g00 + ry * g01) + rx * g02
                c1 = (rz * g10 + ry * g11) + rx * g12
                c2 = (rz * g20 + ry * g21) + rx * g22
                d2 = (c0 * c0 + c1 * c1) + c2 * c2
                x = jnp.maximum(d2, jnp.float32(1e-12))
                i = lax.bitcast_convert_type(x, jnp.int32)
                y = lax.bitcast_convert_type(
                    magic - lax.shift_right_arithmetic(i, 1), jnp.float32)
                hx = half * x
                y = y * (onep5 - hx * y * y)
                y = y * (onep5 - hx * y * y)
                y = y * (onep5 - hx * y * y)
                d = x * y
                ridx = jnp.clip((d * rstep_inv).astype(jnp.int32), 0, nrad_m1)
                dens = plsc.load_gather(rad_v, [ridx])
                mask = ((d2 <= rmax2) & (ozf <= ez)
                        & (oyf <= ey) & (oxf <= ex))
                val = jnp.where(mask, dens * w, jnp.float32(0.0))

                zi = _mod(bz + ozi, gz)
                yi = _mod(by + oyi, gy)
                xi = _mod(bx + oxi, gx)
                flat = (zi * gy + yi) * gx + xi

                val_v[pl.ds(base, _LANES)] = val
                idx_v[j, pl.ds(cc * _LANES, _LANES)] = flat
            return 0

        lax.fori_loop(0, _NCHUNK, row_body, 0)

        descs = [pltpu.async_copy(val_v.at[pl.ds(j * 128, 128)],
                                  grid_sh.at[idx_v.at[j]], sem, add=True)
                 for j in range(_NCHUNK)]
        for dsc in descs:
            dsc.wait()
        return 0

    lax.fori_loop(0, apw, atom_body, 0)

    plsc.subcore_barrier()
    pltpu.sync_copy(grid_sh.at[pl.ds(sid * share, share)],
                    out_hbm.at[cid, pl.ds(sid * share, share)])


def kernel(coordinates, active, occupancies, lmax, radial_densities,
           grid_to_cartesian, out):
    b, n = coordinates.shape[:2]
    bn = b * n
    gz, gy, gx = out.shape[1], out.shape[2], out.shape[3]
    nflat_sc = gz * gy * gx
    nrad = radial_densities.shape[-1]
    nrad_pad = ((nrad + 127) // 128) * 128
    apw = bn // (_NC * _NS)

    f32 = jnp.float32
    coords = coordinates.reshape(bn, 3)
    b_min = jnp.ceil(coords - lmax[None, :]).astype(jnp.int32)
    b_max = jnp.floor(coords + lmax[None, :]).astype(jnp.int32)
    ext = (b_max - b_min).astype(f32)
    w = occupancies.reshape(bn) * active.reshape(bn).astype(f32)

    params = jnp.stack([
        coords[:, 0], coords[:, 1], coords[:, 2],
        b_min[:, 0].astype(f32), b_min[:, 1].astype(f32),
        b_min[:, 2].astype(f32),
        ext[:, 0], ext[:, 1], ext[:, 2],
        w,
    ], axis=1)
    pf = jnp.broadcast_to(params[:, :, None], (bn, _NPF, _LANES))
    pf = jnp.pad(pf.reshape(bn, _NPF * _LANES),
                 ((0, 0), (0, 256 - _NPF * _LANES)))
    pf = pf.reshape(_NC * _NS, apw, 256)

    gb = grid_to_cartesian.astype(jnp.bfloat16).astype(f32)
    m_in = jnp.pad(
        jnp.broadcast_to(gb.reshape(9)[:, None], (9, _LANES)).reshape(144),
        (0, 112))

    t = jnp.arange(_PTS, dtype=jnp.int32)
    oz = jnp.where(t < _L3, t // (_LBOX * _LBOX), _LBOX)
    oy = jnp.where(t < _L3, (t // _LBOX) % _LBOX, _LBOX)
    ox = jnp.where(t < _L3, t % _LBOX, _LBOX)
    tabi = jnp.stack([oz, oy, ox])
    tabf = tabi.astype(f32)

    rad = radial_densities.reshape(bn, nrad)
    rad = jnp.pad(rad, ((0, 0), (0, nrad_pad - nrad)))

    seed = out.reshape(b, nflat_sc)

    sc = functools.partial(
        pl.kernel,
        out_type=jax.ShapeDtypeStruct((b, nflat_sc), f32),
        mesh=plsc.VectorSubcoreMesh(core_axis_name="c", subcore_axis_name="s",
                                    num_cores=_NC, num_subcores=_NS),
        compiler_params=pltpu.CompilerParams(needs_layout_passes=False),
        scratch_types=[
            pltpu.VMEM((3, _PTS), f32),
            pltpu.VMEM((3, _PTS), jnp.int32),
            pltpu.VMEM((apw, 256), f32),
            pltpu.VMEM((256,), f32),
            pltpu.VMEM((nrad_pad,), f32),
            pltpu.VMEM((_PTS,), f32),
            pltpu.VMEM((_NCHUNK, 128), jnp.int32),
            pltpu.VMEM_SHARED((nflat_sc,), f32),
            pltpu.SemaphoreType.DMA,
        ],
    )(functools.partial(_sc_body, apw=apw, nflat_sc=nflat_sc,
                        gz=gz, gy=gy, gx=gx, nrad_m1=nrad - 1))

    res = sc(tabf, tabi, pf, m_in, rad, seed)
    return res.reshape(out.shape)


# speedup vs baseline: 138.5888x; 3.2651x over previous
import functools

import jax
import jax.numpy as jnp
import numpy as np
from jax import lax
from jax.experimental import pallas as pl
from jax.experimental.pallas import tpu as pltpu
from jax.experimental.pallas import tpu_sc as plsc

_RMAX = 3.0
_RSTEP = 0.01
_LBOX = 13
_L3 = _LBOX * _LBOX * _LBOX
_NCHUNK = 18
_PTS = _NCHUNK * 128
_LANES = 16
_NC, _NS = 2, 16
_NPF = 10
_BIG = 1.0e9


def _mod(a, m):
    r = lax.rem(a, m)
    return jnp.where(r < 0, r + m, r)


def _build_bin_table():
    rstep = np.float32(_RSTEP)
    nbins = int(_RMAX / _RSTEP) + 2
    s = np.full(512, np.inf, np.float32)
    s[0] = -np.inf
    f32i = np.float32(np.inf)
    for k in range(1, nbins):
        t = np.float32(k * _RSTEP)
        while np.float32(t / rstep) >= k:
            t = np.nextafter(t, -f32i, dtype=np.float32)
        while np.float32(t / rstep) < k:
            t = np.nextafter(t, f32i, dtype=np.float32)
        x = np.float32(t * t)
        while np.float32(np.sqrt(x)) >= t:
            x = np.nextafter(x, -f32i, dtype=np.float32)
        while np.float32(np.sqrt(x)) < t:
            x = np.nextafter(x, f32i, dtype=np.float32)
        s[k] = x
    return s


_S_TABLE = _build_bin_table()


def _bf16_round(v):
    u = lax.bitcast_convert_type(v, jnp.int32)
    lsb = lax.shift_right_logical(u, 16) & 1
    u = (u + 0x7FFF + lsb) & jnp.int32(-65536)
    return lax.bitcast_convert_type(u, jnp.float32)


def _sc_body(tabg_hbm, pf_hbm, m_hbm, rad_hbm, s_hbm, seed_hbm, out_hbm,
             tabg_v, pf_v, m_v, rad_v, s_v, relb_v, con_v, val_v, idx_v,
             grid_sh, sem,
             *, apw, nflat_sc, gz, gy, gx, nrad_m1):
    cid = lax.axis_index("c")
    sid = lax.axis_index("s")
    wid = cid * _NS + sid
    share = nflat_sc // _NS

    pltpu.sync_copy(seed_hbm.at[cid, pl.ds(sid * share, share)],
                    grid_sh.at[pl.ds(sid * share, share)])

    pltpu.sync_copy(tabg_hbm, tabg_v)
    pltpu.sync_copy(pf_hbm.at[wid], pf_v)
    pltpu.sync_copy(m_hbm, m_v)
    pltpu.sync_copy(s_hbm, s_v)
    plsc.subcore_barrier()

    g00 = m_v[pl.ds(0, _LANES)]
    g01 = m_v[pl.ds(16, _LANES)]
    g02 = m_v[pl.ds(32, _LANES)]
    g10 = m_v[pl.ds(48, _LANES)]
    g11 = m_v[pl.ds(64, _LANES)]
    g12 = m_v[pl.ds(80, _LANES)]
    g20 = m_v[pl.ds(96, _LANES)]
    g21 = m_v[pl.ds(112, _LANES)]
    g22 = m_v[pl.ds(128, _LANES)]
    rmax2 = jnp.full((_LANES,), _RMAX * _RMAX, jnp.float32)
    rstep_inv = jnp.full((_LANES,), jnp.float32(1.0) / jnp.float32(_RSTEP),
                         jnp.float32)
    half = jnp.full((_LANES,), 0.5, jnp.float32)
    onep5 = jnp.full((_LANES,), 1.5, jnp.float32)
    magic = jnp.full((_LANES,), 0x5F3759DF, jnp.int32)
    big = jnp.full((_LANES,), _BIG, jnp.float32)
    dcap = jnp.full((_LANES,), 4.0, jnp.float32)
    iota_i = lax.iota(jnp.int32, 16)
    iota_f = iota_i.astype(jnp.float32)

    def atom_body(a, _):
        gid = wid * apw + a
        pltpu.sync_copy(rad_hbm.at[gid], rad_v)

        cz = pf_v[a, pl.ds(0, _LANES)]
        cy = pf_v[a, pl.ds(16, _LANES)]
        cx = pf_v[a, pl.ds(32, _LANES)]
        bzf = pf_v[a, pl.ds(48, _LANES)]
        byf = pf_v[a, pl.ds(64, _LANES)]
        bxf = pf_v[a, pl.ds(80, _LANES)]
        ez = pf_v[a, pl.ds(96, _LANES)]
        ey = pf_v[a, pl.ds(112, _LANES)]
        ex = pf_v[a, pl.ds(128, _LANES)]
        w = pf_v[a, pl.ds(144, _LANES)]

        rzb = jnp.where(iota_f > ez, big, _bf16_round((bzf + iota_f) - cz))
        ryb = jnp.where(iota_f > ey, big, _bf16_round((byf + iota_f) - cy))
        rxb = jnp.where(iota_f > ex, big, _bf16_round((bxf + iota_f) - cx))
        bz = bzf.astype(jnp.int32)
        by = byf.astype(jnp.int32)
        bx = bxf.astype(jnp.int32)
        zcon = _mod(bz + iota_i, gz) * (gy * gx)
        ycon = _mod(by + iota_i, gy) * gx
        xcon = _mod(bx + iota_i, gx)
        relb_v[pl.ds(0, _LANES)] = rzb
        relb_v[pl.ds(128, _LANES)] = ryb
        relb_v[pl.ds(256, _LANES)] = rxb
        con_v[pl.ds(0, _LANES)] = zcon
        con_v[pl.ds(128, _LANES)] = ycon
        con_v[pl.ds(256, _LANES)] = xcon

        @plsc.parallel_loop(0, _NCHUNK, 1, unroll=2)
        def row_body(j):
            for cc in range(128 // _LANES):
                base = j * 128 + cc * _LANES
                iz = tabg_v[0, pl.ds(base, _LANES)]
                iy = tabg_v[1, pl.ds(base, _LANES)]
                ix = tabg_v[2, pl.ds(base, _LANES)]

                rz = plsc.load_gather(relb_v, [iz])
                ry = plsc.load_gather(relb_v, [iy])
                rx = plsc.load_gather(relb_v, [ix])
                c0 = (rz * g00 + ry * g01) + rx * g02
                c1 = (rz * g10 + ry * g11) + rx * g12
                c2 = (rz * g20 + ry * g21) + rx * g22
                d2 = (c0 * c0 + c1 * c1) + c2 * c2
                x = jnp.maximum(d2, jnp.float32(1e-12))
                i = lax.bitcast_convert_type(x, jnp.int32)
                y = lax.bitcast_convert_type(
                    magic - lax.shift_right_arithmetic(i, 1), jnp.float32)
                hx = half * x
                y = y * (onep5 - hx * y * y)
                y = y * (onep5 - hx * y * y)
                d = jnp.minimum(x * y, dcap)
                k0 = (d * rstep_inv).astype(jnp.int32)
                slo = plsc.load_gather(s_v, [k0])
                shi = plsc.load_gather(s_v, [k0 + 1])
                up = (x >= shi).astype(jnp.int32)
                dn = (x < slo).astype(jnp.int32)
                ridx = jnp.minimum(k0 + up - dn, nrad_m1)
                dens = plsc.load_gather(rad_v, [ridx])
                val = jnp.where(d2 <= rmax2, dens * w, jnp.float32(0.0))

                zc = plsc.load_gather(con_v, [iz])
                yc = plsc.load_gather(con_v, [iy])
                xc = plsc.load_gather(con_v, [ix])
                flat = (zc + yc) + xc

                val_v[pl.ds(base, _LANES)] = val
                idx_v[j, pl.ds(cc * _LANES, _LANES)] = flat

        descs = [pltpu.async_copy(val_v.at[pl.ds(j * 128, 128)],
                                  grid_sh.at[idx_v.at[j]], sem, add=True)
                 for j in range(_NCHUNK)]
        for dsc in descs:
            dsc.wait()
        return 0

    lax.fori_loop(0, apw, atom_body, 0)

    plsc.subcore_barrier()
    pltpu.sync_copy(grid_sh.at[pl.ds(sid * share, share)],
                    out_hbm.at[cid, pl.ds(sid * share, share)])


def kernel(coordinates, active, occupancies, lmax, radial_densities,
           grid_to_cartesian, out):
    b, n = coordinates.shape[:2]
    bn = b * n
    gz, gy, gx = out.shape[1], out.shape[2], out.shape[3]
    nflat_sc = gz * gy * gx
    nrad = radial_densities.shape[-1]
    nrad_pad = ((nrad + 127) // 128) * 128
    apw = bn // (_NC * _NS)

    f32 = jnp.float32
    coords = coordinates.reshape(bn, 3)
    b_min = jnp.ceil(coords - lmax[None, :]).astype(jnp.int32)
    b_max = jnp.floor(coords + lmax[None, :]).astype(jnp.int32)
    ext = (b_max - b_min).astype(f32)
    w = occupancies.reshape(bn) * active.reshape(bn).astype(f32)

    params = jnp.stack([
        coords[:, 0], coords[:, 1], coords[:, 2],
        b_min[:, 0].astype(f32), b_min[:, 1].astype(f32),
        b_min[:, 2].astype(f32),
        ext[:, 0], ext[:, 1], ext[:, 2],
        w,
    ], axis=1)
    pf = jnp.broadcast_to(params[:, :, None], (bn, _NPF, _LANES))
    pf = jnp.pad(pf.reshape(bn, _NPF * _LANES),
                 ((0, 0), (0, 256 - _NPF * _LANES)))
    pf = pf.reshape(_NC * _NS, apw, 256)

    gb = grid_to_cartesian.astype(jnp.bfloat16).astype(f32)
    m_in = jnp.pad(
        jnp.broadcast_to(gb.reshape(9)[:, None], (9, _LANES)).reshape(144),
        (0, 112))

    t = jnp.arange(_PTS, dtype=jnp.int32)
    oz = jnp.where(t < _L3, t // (_LBOX * _LBOX), _LBOX)
    oy = jnp.where(t < _L3, (t // _LBOX) % _LBOX, _LBOX) + 128
    ox = jnp.where(t < _L3, t % _LBOX, _LBOX) + 256
    tabg = jnp.stack([oz, oy, ox])

    rad = radial_densities.reshape(bn, nrad)
    rad = jnp.pad(rad, ((0, 0), (0, nrad_pad - nrad)))

    s_in = jnp.asarray(_S_TABLE)
    seed = out.reshape(b, nflat_sc)

    sc = functools.partial(
        pl.kernel,
        out_type=jax.ShapeDtypeStruct((b, nflat_sc), f32),
        mesh=plsc.VectorSubcoreMesh(core_axis_name="c", subcore_axis_name="s",
                                    num_cores=_NC, num_subcores=_NS),
        compiler_params=pltpu.CompilerParams(needs_layout_passes=False),
        scratch_types=[
            pltpu.VMEM((3, _PTS), jnp.int32),
            pltpu.VMEM((apw, 256), f32),
            pltpu.VMEM((256,), f32),
            pltpu.VMEM((nrad_pad,), f32),
            pltpu.VMEM((512,), f32),
            pltpu.VMEM((384,), f32),
            pltpu.VMEM((384,), jnp.int32),
            pltpu.VMEM((_PTS,), f32),
            pltpu.VMEM((_NCHUNK, 128), jnp.int32),
            pltpu.VMEM_SHARED((nflat_sc,), f32),
            pltpu.SemaphoreType.DMA,
        ],
    )(functools.partial(_sc_body, apw=apw, nflat_sc=nflat_sc,
                        gz=gz, gy=gy, gx=gx, nrad_m1=nrad - 1))

    res = sc(tabg, pf, m_in, rad, s_in, seed)
    return res.reshape(out.shape)


# speedup vs baseline: 161.8407x; 1.1678x over previous
import functools

import jax
import jax.numpy as jnp
import numpy as np
from jax import lax
from jax.experimental import pallas as pl
from jax.experimental.pallas import tpu as pltpu
from jax.experimental.pallas import tpu_sc as plsc

_RMAX = 3.0
_RSTEP = 0.01
_LBOX = 13
_L3 = _LBOX * _LBOX * _LBOX
_NCHUNK = 18
_PTS = _NCHUNK * 128
_LANES = 16
_NC, _NS = 2, 16
_NPF = 10
_BIG = 1.0e9


def _mod(a, m):
    r = lax.rem(a, m)
    return jnp.where(r < 0, r + m, r)


def _build_bin_table():
    rstep = np.float32(_RSTEP)
    nbins = int(_RMAX / _RSTEP) + 2
    s = np.full(512, np.inf, np.float32)
    s[0] = -np.inf
    f32i = np.float32(np.inf)
    for k in range(1, nbins):
        t = np.float32(k * _RSTEP)
        while np.float32(t / rstep) >= k:
            t = np.nextafter(t, -f32i, dtype=np.float32)
        while np.float32(t / rstep) < k:
            t = np.nextafter(t, f32i, dtype=np.float32)
        x = np.float32(t * t)
        while np.float32(np.sqrt(x)) >= t:
            x = np.nextafter(x, -f32i, dtype=np.float32)
        while np.float32(np.sqrt(x)) < t:
            x = np.nextafter(x, f32i, dtype=np.float32)
        s[k] = x
    return s


_S_TABLE = _build_bin_table()


def _bf16_round(v):
    u = lax.bitcast_convert_type(v, jnp.int32)
    lsb = lax.shift_right_logical(u, 16) & 1
    u = (u + 0x7FFF + lsb) & jnp.int32(-65536)
    return lax.bitcast_convert_type(u, jnp.float32)


def _sc_body(tabg_hbm, pf_hbm, m_hbm, rad_hbm, s_hbm, seed_hbm, out_hbm,
             tabg_v, pf_v, m_v, rad_v, s_v, relb_v, con_v, val_v, idx_v,
             grid_sh, sem,
             *, apw, nflat_sc, gz, gy, gx, nrad_m1):
    cid = lax.axis_index("c")
    sid = lax.axis_index("s")
    wid = cid * _NS + sid
    share = nflat_sc // _NS

    pltpu.sync_copy(seed_hbm.at[cid, pl.ds(sid * share, share)],
                    grid_sh.at[pl.ds(sid * share, share)])

    pltpu.sync_copy(tabg_hbm, tabg_v)
    pltpu.sync_copy(pf_hbm.at[wid], pf_v)
    pltpu.sync_copy(m_hbm, m_v)
    pltpu.sync_copy(s_hbm, s_v)
    plsc.subcore_barrier()

    g00 = m_v[pl.ds(0, _LANES)]
    g01 = m_v[pl.ds(16, _LANES)]
    g02 = m_v[pl.ds(32, _LANES)]
    g10 = m_v[pl.ds(48, _LANES)]
    g11 = m_v[pl.ds(64, _LANES)]
    g12 = m_v[pl.ds(80, _LANES)]
    g20 = m_v[pl.ds(96, _LANES)]
    g21 = m_v[pl.ds(112, _LANES)]
    g22 = m_v[pl.ds(128, _LANES)]
    rmax2 = jnp.full((_LANES,), _RMAX * _RMAX, jnp.float32)
    rstep_inv = jnp.full((_LANES,), jnp.float32(1.0) / jnp.float32(_RSTEP),
                         jnp.float32)
    half = jnp.full((_LANES,), 0.5, jnp.float32)
    onep5 = jnp.full((_LANES,), 1.5, jnp.float32)
    magic = jnp.full((_LANES,), 0x5F3759DF, jnp.int32)
    big = jnp.full((_LANES,), _BIG, jnp.float32)
    dcap = jnp.full((_LANES,), 4.0, jnp.float32)
    iota_i = lax.iota(jnp.int32, 16)
    iota_f = iota_i.astype(jnp.float32)

    def atom_body(a, _):
        gid = wid * apw + a
        pltpu.sync_copy(rad_hbm.at[gid], rad_v)

        cz = pf_v[a, pl.ds(0, _LANES)]
        cy = pf_v[a, pl.ds(16, _LANES)]
        cx = pf_v[a, pl.ds(32, _LANES)]
        bzf = pf_v[a, pl.ds(48, _LANES)]
        byf = pf_v[a, pl.ds(64, _LANES)]
        bxf = pf_v[a, pl.ds(80, _LANES)]
        ez = pf_v[a, pl.ds(96, _LANES)]
        ey = pf_v[a, pl.ds(112, _LANES)]
        ex = pf_v[a, pl.ds(128, _LANES)]
        w = pf_v[a, pl.ds(144, _LANES)]

        rzb = jnp.where(iota_f > ez, big, _bf16_round((bzf + iota_f) - cz))
        ryb = jnp.where(iota_f > ey, big, _bf16_round((byf + iota_f) - cy))
        rxb = jnp.where(iota_f > ex, big, _bf16_round((bxf + iota_f) - cx))
        bz = bzf.astype(jnp.int32)
        by = byf.astype(jnp.int32)
        bx = bxf.astype(jnp.int32)
        zcon = _mod(bz + iota_i, gz) * (gy * gx)
        ycon = _mod(by + iota_i, gy) * gx
        xcon = _mod(bx + iota_i, gx)
        relb_v[pl.ds(0, _LANES)] = rzb
        relb_v[pl.ds(128, _LANES)] = ryb
        relb_v[pl.ds(256, _LANES)] = rxb
        con_v[pl.ds(0, _LANES)] = zcon
        con_v[pl.ds(128, _LANES)] = ycon
        con_v[pl.ds(256, _LANES)] = xcon

        @plsc.parallel_loop(0, _NCHUNK, 1, unroll=3)
        def row_body(j):
            for cc in range(128 // _LANES):
                base = j * 128 + cc * _LANES
                iz = tabg_v[0, pl.ds(base, _LANES)]
                iy = tabg_v[1, pl.ds(base, _LANES)]
                ix = tabg_v[2, pl.ds(base, _LANES)]

                rz = plsc.load_gather(relb_v, [iz])
                ry = plsc.load_gather(relb_v, [iy])
                rx = plsc.load_gather(relb_v, [ix])
                c0 = (rz * g00 + ry * g01) + rx * g02
                c1 = (rz * g10 + ry * g11) + rx * g12
                c2 = (rz * g20 + ry * g21) + rx * g22
                d2 = (c0 * c0 + c1 * c1) + c2 * c2
                x = jnp.maximum(d2, jnp.float32(1e-12))
                i = lax.bitcast_convert_type(x, jnp.int32)
                y = lax.bitcast_convert_type(
                    magic - lax.shift_right_arithmetic(i, 1), jnp.float32)
                hx = half * x
                y = y * (onep5 - hx * y * y)
                y = y * (onep5 - hx * y * y)
                d = jnp.minimum(x * y, dcap)
                k0 = (d * rstep_inv).astype(jnp.int32)
                slo = plsc.load_gather(s_v, [k0])
                shi = plsc.load_gather(s_v, [k0 + 1])
                up = (x >= shi).astype(jnp.int32)
                dn = (x < slo).astype(jnp.int32)
                ridx = jnp.minimum(k0 + up - dn, nrad_m1)
                dens = plsc.load_gather(rad_v, [ridx])
                val = jnp.where(d2 <= rmax2, dens * w, jnp.float32(0.0))

                zc = plsc.load_gather(con_v, [iz])
                yc = plsc.load_gather(con_v, [iy])
                xc = plsc.load_gather(con_v, [ix])
                flat = (zc + yc) + xc

                val_v[pl.ds(base, _LANES)] = val
                idx_v[j, pl.ds(cc * _LANES, _LANES)] = flat

        descs = [pltpu.async_copy(val_v.at[pl.ds(j * 128, 128)],
                                  grid_sh.at[idx_v.at[j]], sem, add=True)
                 for j in range(_NCHUNK)]
        for dsc in descs:
            dsc.wait()
        return 0

    lax.fori_loop(0, apw, atom_body, 0)

    plsc.subcore_barrier()
    pltpu.sync_copy(grid_sh.at[pl.ds(sid * share, share)],
                    out_hbm.at[cid, pl.ds(sid * share, share)])


def kernel(coordinates, active, occupancies, lmax, radial_densities,
           grid_to_cartesian, out):
    b, n = coordinates.shape[:2]
    bn = b * n
    gz, gy, gx = out.shape[1], out.shape[2], out.shape[3]
    nflat_sc = gz * gy * gx
    nrad = radial_densities.shape[-1]
    nrad_pad = ((nrad + 127) // 128) * 128
    apw = bn // (_NC * _NS)

    f32 = jnp.float32
    coords = coordinates.reshape(bn, 3)
    b_min = jnp.ceil(coords - lmax[None, :]).astype(jnp.int32)
    b_max = jnp.floor(coords + lmax[None, :]).astype(jnp.int32)
    ext = (b_max - b_min).astype(f32)
    w = occupancies.reshape(bn) * active.reshape(bn).astype(f32)

    params = jnp.stack([
        coords[:, 0], coords[:, 1], coords[:, 2],
        b_min[:, 0].astype(f32), b_min[:, 1].astype(f32),
        b_min[:, 2].astype(f32),
        ext[:, 0], ext[:, 1], ext[:, 2],
        w,
    ], axis=1)
    pf = jnp.broadcast_to(params[:, :, None], (bn, _NPF, _LANES))
    pf = jnp.pad(pf.reshape(bn, _NPF * _LANES),
                 ((0, 0), (0, 256 - _NPF * _LANES)))
    pf = pf.reshape(_NC * _NS, apw, 256)

    gb = grid_to_cartesian.astype(jnp.bfloat16).astype(f32)
    m_in = jnp.pad(
        jnp.broadcast_to(gb.reshape(9)[:, None], (9, _LANES)).reshape(144),
        (0, 112))

    t = jnp.arange(_PTS, dtype=jnp.int32)
    oz = jnp.where(t < _L3, t // (_LBOX * _LBOX), _LBOX)
    oy = jnp.where(t < _L3, (t // _LBOX) % _LBOX, _LBOX) + 128
    ox = jnp.where(t < _L3, t % _LBOX, _LBOX) + 256
    tabg = jnp.stack([oz, oy, ox])

    rad = radial_densities.reshape(bn, nrad)
    rad = jnp.pad(rad, ((0, 0), (0, nrad_pad - nrad)))

    s_in = jnp.asarray(_S_TABLE)
    seed = out.reshape(b, nflat_sc)

    sc = functools.partial(
        pl.kernel,
        out_type=jax.ShapeDtypeStruct((b, nflat_sc), f32),
        mesh=plsc.VectorSubcoreMesh(core_axis_name="c", subcore_axis_name="s",
                                    num_cores=_NC, num_subcores=_NS),
        compiler_params=pltpu.CompilerParams(needs_layout_passes=False),
        scratch_types=[
            pltpu.VMEM((3, _PTS), jnp.int32),
            pltpu.VMEM((apw, 256), f32),
            pltpu.VMEM((256,), f32),
            pltpu.VMEM((nrad_pad,), f32),
            pltpu.VMEM((512,), f32),
            pltpu.VMEM((384,), f32),
            pltpu.VMEM((384,), jnp.int32),
            pltpu.VMEM((_PTS,), f32),
            pltpu.VMEM((_NCHUNK, 128), jnp.int32),
            pltpu.VMEM_SHARED((nflat_sc,), f32),
            pltpu.SemaphoreType.DMA,
        ],
    )(functools.partial(_sc_body, apw=apw, nflat_sc=nflat_sc,
                        gz=gz, gy=gy, gx=gx, nrad_m1=nrad - 1))

    res = sc(tabg, pf, m_in, rad, s_in, seed)
    return res.reshape(out.shape)
